# baseline (speedup 1.0000x reference)
import jax
import jax.numpy as jnp
from jax.experimental import pallas as pl
from jax.experimental.pallas import tpu as pltpu

P = 16
E = 16
C = 96
FF = 96
EPS = 1e-6


def _moe_row(x_ref, wr_ref, w1_ref, w2_ref, y_ref,
             rb_ref, xt_ref, ot_ref, yb_ref, isem, osem):
    wr = wr_ref[:]
    w = x_ref.shape[-1]
    wp = w // P

    in_cp = [pltpu.make_async_copy(x_ref.at[0, :, r, :], rb_ref.at[r],
                                   isem.at[r]) for r in range(P)]
    for cp in in_cp:
        cp.start()

    s = jnp.zeros((C, w), dtype=jnp.float32)
    for r in range(P):
        in_cp[r].wait()
        xr = rb_ref[r]
        ms = jnp.mean(xr * xr, axis=0, keepdims=True)
        z = xr * jax.lax.rsqrt(ms + EPS)
        xt_ref[r] = z.T
        s = s + z

    wi = jax.lax.broadcasted_iota(jnp.int32, (w, wp), 0)
    ji = jax.lax.broadcasted_iota(jnp.int32, (w, wp), 1)
    seg = jnp.where(wi // P == ji, 1.0, 0.0)
    pooled = jax.lax.dot_general(
        s, seg, (((1,), (0,)), ((), ())),
        preferred_element_type=jnp.float32)
    logits = jax.lax.dot_general(
        pooled, wr, (((0,), (0,)), ((), ())),
        preferred_element_type=jnp.float32)
    lmax = jnp.max(logits, axis=1, keepdims=True)
    lane = jax.lax.broadcasted_iota(jnp.int32, (wp, E), 1)
    idx = jnp.min(jnp.where(logits >= lmax, lane, E), axis=1,
                  keepdims=True)

    for j in range(wp):
        e = idx[j, 0]
        xp = xt_ref[:, j * P:(j + 1) * P, :].reshape(P * P, C)
        w1 = w1_ref[e]
        w2 = w2_ref[e]
        h = jax.lax.dot_general(
            xp, w1, (((1,), (0,)), ((), ())),
            preferred_element_type=jnp.float32)
        h = h * jax.nn.sigmoid(h)
        o = jax.lax.dot_general(
            h, w2, (((1,), (0,)), ((), ())),
            preferred_element_type=jnp.float32)
        ot_ref[:, j * P:(j + 1) * P, :] = o.reshape(P, P, C)

    out_cp = []
    for r in range(P):
        yb_ref[r] = rb_ref[r] + ot_ref[r].T
        cp = pltpu.make_async_copy(yb_ref.at[r], y_ref.at[0, :, r, :],
                                   osem.at[r])
        cp.start()
        out_cp.append(cp)
    for cp in out_cp:
        cp.wait()


def kernel(x, g, Wr, W1, W2):
    B, Cc, H, W = x.shape
    Hp = H // P

    wrg = g[:, None] * Wr
    w1g = g[None, :, None] * W1

    y = pl.pallas_call(
        _moe_row,
        grid=(B, Hp),
        in_specs=[
            pl.BlockSpec((1, Cc, P, W), lambda b, i: (b, 0, i, 0)),
            pl.BlockSpec((Cc, E), lambda b, i: (0, 0)),
            pl.BlockSpec((E, Cc, FF), lambda b, i: (0, 0, 0)),
            pl.BlockSpec((E, FF, Cc), lambda b, i: (0, 0, 0)),
        ],
        out_specs=pl.BlockSpec((1, Cc, P, W), lambda b, i: (b, 0, i, 0)),
        out_shape=jax.ShapeDtypeStruct((B, Cc, H, W), x.dtype),
        scratch_shapes=[
            pltpu.VMEM((P, Cc, W), jnp.float32),
            pltpu.VMEM((P, W, Cc), jnp.float32),
            pltpu.VMEM((P, W, Cc), jnp.float32),
            pltpu.VMEM((P, Cc, W), jnp.float32),
            pltpu.SemaphoreType.DMA((P,)),
            pltpu.SemaphoreType.DMA((P,)),
        ],
    )(x, wrg, w1g, W2)

    return y


# speedup vs baseline: 1.5663x; 1.5663x over previous
import jax
import jax.numpy as jnp
from jax.experimental import pallas as pl
from jax.experimental.pallas import tpu as pltpu

P = 16
E = 16
C = 96
FF = 96
EPS = 1e-6
RB = 2


def _moe_row(x_ref, wr_ref, w1_ref, w2_ref, y_ref, xt_ref, ot_ref):
    wr = wr_ref[:]
    w = x_ref.shape[-1]
    wp = w // P

    ss = []
    for q in range(RB):
        s = jnp.zeros((C, w), dtype=jnp.float32)
        for r in range(P):
            xr = x_ref[0, :, q * P + r, :]
            ms = jnp.mean(xr * xr, axis=0, keepdims=True)
            z = xr * jax.lax.rsqrt(ms + EPS)
            xt_ref[q * P + r] = z.T
            s = s + z
        ss.append(s)
    s2 = jnp.concatenate(ss, axis=1)

    nt = RB * wp
    wi = jax.lax.broadcasted_iota(jnp.int32, (RB * w, nt), 0)
    ji = jax.lax.broadcasted_iota(jnp.int32, (RB * w, nt), 1)
    seg = jnp.where(wi // P == ji, 1.0, 0.0)
    pooled = jax.lax.dot_general(
        s2, seg, (((1,), (0,)), ((), ())),
        preferred_element_type=jnp.float32)
    logits = jax.lax.dot_general(
        pooled, wr, (((0,), (0,)), ((), ())),
        preferred_element_type=jnp.float32)
    lmax = jnp.max(logits, axis=1, keepdims=True)
    lane = jax.lax.broadcasted_iota(jnp.int32, (nt, E), 1)
    idx = jnp.min(jnp.where(logits >= lmax, lane, E), axis=1,
                  keepdims=True)

    for t in range(nt):
        q, j = divmod(t, wp)
        e = idx[t, 0]
        xp = xt_ref[q * P:(q + 1) * P,
                    j * P:(j + 1) * P, :].reshape(P * P, C)
        w1 = w1_ref[e]
        w2 = w2_ref[e]
        h = jax.lax.dot_general(
            xp, w1, (((1,), (0,)), ((), ())),
            preferred_element_type=jnp.float32)
        h = h * jax.nn.sigmoid(h)
        o = jax.lax.dot_general(
            h, w2, (((1,), (0,)), ((), ())),
            preferred_element_type=jnp.float32)
        ot_ref[q * P:(q + 1) * P,
               j * P:(j + 1) * P, :] = o.reshape(P, P, C)

    for r in range(RB * P):
        y_ref[0, :, r, :] = x_ref[0, :, r, :] + ot_ref[r].T


def kernel(x, g, Wr, W1, W2):
    B, Cc, H, W = x.shape
    Hb = H // (RB * P)

    wrg = g[:, None] * Wr
    w1g = g[None, :, None] * W1

    y = pl.pallas_call(
        _moe_row,
        grid=(B, Hb),
        in_specs=[
            pl.BlockSpec((1, Cc, RB * P, W), lambda b, i: (b, 0, i, 0)),
            pl.BlockSpec((Cc, E), lambda b, i: (0, 0)),
            pl.BlockSpec((E, Cc, FF), lambda b, i: (0, 0, 0)),
            pl.BlockSpec((E, FF, Cc), lambda b, i: (0, 0, 0)),
        ],
        out_specs=pl.BlockSpec((1, Cc, RB * P, W), lambda b, i: (b, 0, i, 0)),
        out_shape=jax.ShapeDtypeStruct((B, Cc, H, W), x.dtype),
        scratch_shapes=[
            pltpu.VMEM((RB * P, W, Cc), jnp.float32),
            pltpu.VMEM((RB * P, W, Cc), jnp.float32),
        ],
    )(x, wrg, w1g, W2)

    return y
